# baseline (speedup 1.0000x reference)
import functools

import jax
import jax.numpy as jnp
from jax import lax
from jax.experimental import pallas as pl
from jax.experimental.pallas import tpu as pltpu
from jax.experimental.pallas import tpu_sc as plsc

NUM_ROUND = 4
N = 10000
E = 320000
D = 128

NS = 16
EPT = E // NS
K = 80
CPT = EPT // K
NHALF = CPT // 2
ROW_GRAN = 80
NCH = N // ROW_GRAN
CH_BASE = NCH // NS
CH_EXTRA = NCH % NS

BR = 2000
G = (2 * N) // BR


def _sc_body(msg_hbm, src_hbm, dst_hbm, dstp_hbm,
             agg_hbm,
             g0, g1, g2, g3, s0, s1, s2, s3, r0, r1, r2, r3, acc,
             i0, i1, i2, i3, q0, q1, q2, q3, w0, w1, w2, w3):
    cid = lax.axis_index("c")
    sid = lax.axis_index("s")
    n_ch = CH_BASE + jnp.where(sid < CH_EXTRA, 1, 0)
    ch0 = sid * CH_BASE + jnp.minimum(sid, CH_EXTRA)
    gidx = [g0, g1, g2, g3]
    sidx = [s0, s1, s2, s3]
    rows = [r0, r1, r2, r3]
    isem = [i0, i1, i2, i3]
    gsem = [q0, q1, q2, q3]
    ssem = [w0, w1, w2, w3]

    def zrow(i, c):
        for j in range(8):
            r0[i, pl.ds(j * 16, 16)] = jnp.zeros((16,), jnp.float32)
        return c
    lax.fori_loop(0, ROW_GRAN, zrow, 0)

    def zero_chunk(i, c):
        off = pl.multiple_of((ch0 + i) * ROW_GRAN, 8)
        pltpu.sync_copy(r0, acc.at[pl.ds(off, ROW_GRAN)])
        return c
    lax.fori_loop(0, n_ch, zero_chunk, 0)
    plsc.subcore_barrier()

    def do_dir(g_hbm, s_hbm, out_off):
        ebase = sid * EPT

        def IS(b, i):
            st = ebase + i * K
            pltpu.async_copy(g_hbm.at[pl.ds(st, K)], gidx[b], isem[b])
            pltpu.async_copy(s_hbm.at[pl.ds(st, K)], sidx[b], isem[b])

        def IW(b):
            pltpu.make_async_copy(g_hbm.at[pl.ds(0, K)], gidx[b], isem[b]).wait()
            pltpu.make_async_copy(s_hbm.at[pl.ds(0, K)], sidx[b], isem[b]).wait()

        def GS(b):
            pltpu.async_copy(msg_hbm.at[gidx[b]], rows[b], gsem[b])

        def GW(b):
            pltpu.make_async_copy(msg_hbm.at[gidx[b]], rows[b], gsem[b]).wait()

        def SS(b):
            pltpu.async_copy(rows[b], acc.at[sidx[b]], ssem[b], add=True)

        def SW(b):
            pltpu.make_async_copy(rows[b], acc.at[sidx[b]], ssem[b]).wait()

        IS(0, 0)
        IS(1, 1)
        IS(2, 2)
        IW(0)
        GS(0)
        GW(0); SS(0); IS(3, 3); IW(1); GS(1)
        GW(1); SS(1); IW(2); GS(2)

        def quad(j, c):
            t = 4 * j + 2
            for p in range(4):
                bt = (2 + p) % 4
                bn = (3 + p) % 4
                bo = p % 4
                GW(bt)
                SS(bt)
                SW(bo)
                IS(bo, jnp.minimum(t + p + 2, CPT - 1))
                IW(bn)
                GS(bn)
            return c
        lax.fori_loop(0, 62, quad, 0)
        SW(0)
        SW(1)
        GW(2)
        IW(3)
        plsc.subcore_barrier()

        def out_chunk(i, c):
            off = pl.multiple_of((ch0 + i) * ROW_GRAN, 8)
            pltpu.sync_copy(acc.at[pl.ds(off, ROW_GRAN)],
                            agg_hbm.at[pl.ds(out_off + off, ROW_GRAN)])
            return c
        lax.fori_loop(0, n_ch, out_chunk, 0)

    @pl.when(cid == 0)
    def _():
        do_dir(src_hbm, dst_hbm, N)
    @pl.when(cid == 1)
    def _():
        do_dir(dstp_hbm, src_hbm, 0)


@functools.cache
def _sc_segsum():
    return pl.kernel(
        _sc_body,
        out_type=jax.ShapeDtypeStruct((2 * N, D), jnp.float32),
        mesh=plsc.VectorSubcoreMesh(core_axis_name="c", subcore_axis_name="s"),
        scratch_types=(
            [pltpu.VMEM((K,), jnp.int32) for _ in range(8)]
            + [pltpu.VMEM((K, D), jnp.float32) for _ in range(4)]
            + [pltpu.VMEM_SHARED((N, D), jnp.float32)]
            + [pltpu.SemaphoreType.DMA for _ in range(12)]
        ),
    )


def _msg_body(x_ref, w1_ref, b1_ref, w2_ref, b2_ref, o_ref):
    h = jnp.maximum(
        jnp.dot(x_ref[...], w1_ref[0], preferred_element_type=jnp.float32)
        + b1_ref[0], 0.0)
    o_ref[...] = (jnp.dot(h, w2_ref[0], preferred_element_type=jnp.float32)
                  + b2_ref[0])


_W_SPEC = pl.BlockSpec((1, D, D), lambda i: (i * 2 // G, 0, 0))
_B_SPEC = pl.BlockSpec((1, 1, D), lambda i: (i * 2 // G, 0, 0))
_X_SPEC = pl.BlockSpec((BR, D), lambda i: (i, 0))

_tc_msg = pl.pallas_call(
    _msg_body,
    grid=(G,),
    in_specs=[_X_SPEC, _W_SPEC, _B_SPEC, _W_SPEC, _B_SPEC],
    out_specs=_X_SPEC,
    out_shape=jax.ShapeDtypeStruct((2 * N, D), jnp.float32),
)


def _upd_body(a_ref, x_ref, w1a_ref, w1b_ref, b1_ref, w2_ref, b2_ref, o_ref):
    h = jnp.maximum(
        jnp.dot(a_ref[...], w1a_ref[0], preferred_element_type=jnp.float32)
        + jnp.dot(x_ref[...], w1b_ref[0], preferred_element_type=jnp.float32)
        + b1_ref[0], 0.0)
    o_ref[...] = (jnp.dot(h, w2_ref[0], preferred_element_type=jnp.float32)
                  + b2_ref[0])


_tc_upd = pl.pallas_call(
    _upd_body,
    grid=(G,),
    in_specs=[_X_SPEC, _X_SPEC, _W_SPEC, _W_SPEC, _B_SPEC, _W_SPEC, _B_SPEC],
    out_specs=_X_SPEC,
    out_shape=jax.ShapeDtypeStruct((2 * N, D), jnp.float32),
)


def kernel(v_emb, c_emb, edge_index,
           vmsg_W1, vmsg_b1, vmsg_W2, vmsg_b2,
           cmsg_W1, cmsg_b1, cmsg_W2, cmsg_b2,
           vupd_W1, vupd_b1, vupd_W2, vupd_b2,
           cupd_W1, cupd_b1, cupd_W2, cupd_b2):
    src = edge_index[0].astype(jnp.int32)
    dst = edge_index[1].astype(jnp.int32)
    dstp = dst + N

    msg_W1 = jnp.stack([vmsg_W1, cmsg_W1])
    msg_b1 = jnp.stack([vmsg_b1, cmsg_b1])[:, None, :]
    msg_W2 = jnp.stack([vmsg_W2, cmsg_W2])
    msg_b2 = jnp.stack([vmsg_b2, cmsg_b2])[:, None, :]
    upd_W1a = jnp.stack([vupd_W1[:D], cupd_W1[:D]])
    upd_W1b = jnp.stack([vupd_W1[D:], cupd_W1[D:]])
    upd_b1 = jnp.stack([vupd_b1, cupd_b1])[:, None, :]
    upd_W2 = jnp.stack([vupd_W2, cupd_W2])
    upd_b2 = jnp.stack([vupd_b2, cupd_b2])[:, None, :]

    emb = jnp.concatenate([v_emb, c_emb], axis=0)
    for _ in range(NUM_ROUND):
        msg = _tc_msg(emb, msg_W1, msg_b1, msg_W2, msg_b2)
        agg = _sc_segsum()(msg, src, dst, dstp)
        emb = _tc_upd(agg, emb, upd_W1a, upd_W1b, upd_b1, upd_W2, upd_b2)
    return (emb[:N], emb[N:])


# speedup vs baseline: 1.5088x; 1.5088x over previous
import functools

import jax
import jax.numpy as jnp
from jax import lax
from jax.experimental import pallas as pl
from jax.experimental.pallas import tpu as pltpu
from jax.experimental.pallas import tpu_sc as plsc

NUM_ROUND = 4
N = 10000
E = 320000
D = 128

NS = 16
EPT = E // NS
K = 80
CPT = EPT // K
NHALF = CPT // 2
ROW_GRAN = 80
NCH = N // ROW_GRAN
CH_BASE = NCH // NS
CH_EXTRA = NCH % NS

BR = 2000
G = (2 * N) // BR


def _sc_body(msg_hbm, src_hbm, dst_hbm, dstp_hbm,
             agg_hbm,
             g0, g1, g2, g3, g4, g5, g6, g7,
             s0, s1, s2, s3, s4, s5, s6, s7,
             r0, r1, r2, r3, acc,
             i0, i1, i2, i3, i4, i5, i6, i7,
             q0, q1, q2, q3, w0, w1, w2, w3):
    cid = lax.axis_index("c")
    sid = lax.axis_index("s")
    n_ch = CH_BASE + jnp.where(sid < CH_EXTRA, 1, 0)
    ch0 = sid * CH_BASE + jnp.minimum(sid, CH_EXTRA)
    gidx = [g0, g1, g2, g3, g4, g5, g6, g7]
    sidx = [s0, s1, s2, s3, s4, s5, s6, s7]
    rows = [r0, r1, r2, r3]
    isem = [i0, i1, i2, i3, i4, i5, i6, i7]
    gsem = [q0, q1, q2, q3]
    ssem = [w0, w1, w2, w3]

    def zrow(i, c):
        for j in range(8):
            r0[i, pl.ds(j * 16, 16)] = jnp.zeros((16,), jnp.float32)
        return c
    lax.fori_loop(0, ROW_GRAN, zrow, 0)

    def zero_chunk(i, c):
        off = pl.multiple_of((ch0 + i) * ROW_GRAN, 8)
        pltpu.sync_copy(r0, acc.at[pl.ds(off, ROW_GRAN)])
        return c
    lax.fori_loop(0, n_ch, zero_chunk, 0)
    plsc.subcore_barrier()

    def do_dir(g_hbm, s_hbm, out_off):
        ebase = sid * EPT

        def IS(e, i):
            st = ebase + i * K
            pltpu.async_copy(g_hbm.at[pl.ds(st, K)], gidx[e], isem[e])
            pltpu.async_copy(s_hbm.at[pl.ds(st, K)], sidx[e], isem[e])

        def IW(e):
            pltpu.make_async_copy(g_hbm.at[pl.ds(0, K)], gidx[e], isem[e]).wait()
            pltpu.make_async_copy(s_hbm.at[pl.ds(0, K)], sidx[e], isem[e]).wait()

        def GS(b, e):
            pltpu.async_copy(msg_hbm.at[gidx[e]], rows[b], gsem[b])

        def GW(b, e):
            pltpu.make_async_copy(msg_hbm.at[gidx[e]], rows[b], gsem[b]).wait()

        def SS(b, e):
            pltpu.async_copy(rows[b], acc.at[sidx[e]], ssem[b], add=True)

        def SW(b, e):
            pltpu.make_async_copy(rows[b], acc.at[sidx[e]], ssem[b]).wait()

        def step(t, p4, p8, clamp=False):
            SW((p4 + 2) % 4, (p8 + 6) % 8)
            c_pre = jnp.minimum(t + 4, CPT - 1) if clamp else t + 4
            IS((p8 + 4) % 8, c_pre)
            IW((p8 + 2) % 8)
            GS((p4 + 2) % 4, (p8 + 2) % 8)
            GW(p4, p8)
            SS(p4, p8)

        IS(0, 0); IS(1, 1); IS(2, 2); IS(3, 3)
        IW(0); GS(0, 0)
        IW(1); GS(1, 1)
        IS(4, 4); IW(2); GS(2, 2); GW(0, 0); SS(0, 0)
        IS(5, 5); IW(3); GS(3, 3); GW(1, 1); SS(1, 1)
        SW(0, 0); IS(6, 6); IW(4); GS(0, 4); GW(2, 2); SS(2, 2)
        SW(1, 1); IS(7, 7); IW(5); GS(1, 5); GW(3, 3); SS(3, 3)

        def octet(j, c):
            t = 8 * j + 4
            for p in range(8):
                step(t + p, p % 4, (4 + p) % 8)
            return c
        lax.fori_loop(0, 30, octet, 0)
        for u in range(244, 250):
            step(u, u % 4, u % 8, clamp=True)
        SW(0, 0); SW(1, 1)
        GW(2, 2); GW(3, 3)
        IW(4); IW(5)
        plsc.subcore_barrier()

        def out_chunk(i, c):
            off = pl.multiple_of((ch0 + i) * ROW_GRAN, 8)
            pltpu.sync_copy(acc.at[pl.ds(off, ROW_GRAN)],
                            agg_hbm.at[pl.ds(out_off + off, ROW_GRAN)])
            return c
        lax.fori_loop(0, n_ch, out_chunk, 0)

    @pl.when(cid == 0)
    def _():
        do_dir(src_hbm, dst_hbm, N)
    @pl.when(cid == 1)
    def _():
        do_dir(dstp_hbm, src_hbm, 0)


@functools.cache
def _sc_segsum():
    return pl.kernel(
        _sc_body,
        out_type=jax.ShapeDtypeStruct((2 * N, D), jnp.float32),
        mesh=plsc.VectorSubcoreMesh(core_axis_name="c", subcore_axis_name="s"),
        scratch_types=(
            [pltpu.VMEM((K,), jnp.int32) for _ in range(16)]
            + [pltpu.VMEM((K, D), jnp.float32) for _ in range(4)]
            + [pltpu.VMEM_SHARED((N, D), jnp.float32)]
            + [pltpu.SemaphoreType.DMA for _ in range(16)]
        ),
    )


def _msg_body(x_ref, w1_ref, b1_ref, w2_ref, b2_ref, o_ref):
    h = jnp.maximum(
        jnp.dot(x_ref[...], w1_ref[0], preferred_element_type=jnp.float32)
        + b1_ref[0], 0.0)
    o_ref[...] = (jnp.dot(h, w2_ref[0], preferred_element_type=jnp.float32)
                  + b2_ref[0])


_W_SPEC = pl.BlockSpec((1, D, D), lambda i: (i * 2 // G, 0, 0))
_B_SPEC = pl.BlockSpec((1, 1, D), lambda i: (i * 2 // G, 0, 0))
_X_SPEC = pl.BlockSpec((BR, D), lambda i: (i, 0))

_tc_msg = pl.pallas_call(
    _msg_body,
    grid=(G,),
    in_specs=[_X_SPEC, _W_SPEC, _B_SPEC, _W_SPEC, _B_SPEC],
    out_specs=_X_SPEC,
    out_shape=jax.ShapeDtypeStruct((2 * N, D), jnp.float32),
)


def _upd_body(a_ref, x_ref, w1a_ref, w1b_ref, b1_ref, w2_ref, b2_ref, o_ref):
    h = jnp.maximum(
        jnp.dot(a_ref[...], w1a_ref[0], preferred_element_type=jnp.float32)
        + jnp.dot(x_ref[...], w1b_ref[0], preferred_element_type=jnp.float32)
        + b1_ref[0], 0.0)
    o_ref[...] = (jnp.dot(h, w2_ref[0], preferred_element_type=jnp.float32)
                  + b2_ref[0])


_tc_upd = pl.pallas_call(
    _upd_body,
    grid=(G,),
    in_specs=[_X_SPEC, _X_SPEC, _W_SPEC, _W_SPEC, _B_SPEC, _W_SPEC, _B_SPEC],
    out_specs=_X_SPEC,
    out_shape=jax.ShapeDtypeStruct((2 * N, D), jnp.float32),
)


def kernel(v_emb, c_emb, edge_index,
           vmsg_W1, vmsg_b1, vmsg_W2, vmsg_b2,
           cmsg_W1, cmsg_b1, cmsg_W2, cmsg_b2,
           vupd_W1, vupd_b1, vupd_W2, vupd_b2,
           cupd_W1, cupd_b1, cupd_W2, cupd_b2):
    src = edge_index[0].astype(jnp.int32)
    dst = edge_index[1].astype(jnp.int32)
    dstp = dst + N

    msg_W1 = jnp.stack([vmsg_W1, cmsg_W1])
    msg_b1 = jnp.stack([vmsg_b1, cmsg_b1])[:, None, :]
    msg_W2 = jnp.stack([vmsg_W2, cmsg_W2])
    msg_b2 = jnp.stack([vmsg_b2, cmsg_b2])[:, None, :]
    upd_W1a = jnp.stack([vupd_W1[:D], cupd_W1[:D]])
    upd_W1b = jnp.stack([vupd_W1[D:], cupd_W1[D:]])
    upd_b1 = jnp.stack([vupd_b1, cupd_b1])[:, None, :]
    upd_W2 = jnp.stack([vupd_W2, cupd_W2])
    upd_b2 = jnp.stack([vupd_b2, cupd_b2])[:, None, :]

    emb = jnp.concatenate([v_emb, c_emb], axis=0)
    for _ in range(NUM_ROUND):
        msg = _tc_msg(emb, msg_W1, msg_b1, msg_W2, msg_b2)
        agg = _sc_segsum()(msg, src, dst, dstp)
        emb = _tc_upd(agg, emb, upd_W1a, upd_W1b, upd_b1, upd_W2, upd_b2)
    return (emb[:N], emb[N:])


# speedup vs baseline: 1.5599x; 1.0339x over previous
import functools

import jax
import jax.numpy as jnp
from jax import lax
from jax.experimental import pallas as pl
from jax.experimental.pallas import tpu as pltpu
from jax.experimental.pallas import tpu_sc as plsc

NUM_ROUND = 4
N = 10000
E = 320000
D = 128

NS = 16
EPT = E // NS
K = 80
CPT = EPT // K
NHALF = CPT // 2
ROW_GRAN = 80
NCH = N // ROW_GRAN
CH_BASE = NCH // NS
CH_EXTRA = NCH % NS

BR = 2000
G = (2 * N) // BR


def _sc_body(msg_hbm, src_hbm, dst_hbm, dstp_hbm,
             agg_hbm,
             g0, g1, g2, g3, g4, g5, g6, g7,
             s0, s1, s2, s3, s4, s5, s6, s7,
             r0, r1, r2, r3, acc,
             i0, i1, i2, i3, i4, i5, i6, i7,
             q0, q1, q2, q3, w0, w1, w2, w3):
    cid = lax.axis_index("c")
    sid = lax.axis_index("s")
    n_ch = CH_BASE + jnp.where(sid < CH_EXTRA, 1, 0)
    ch0 = sid * CH_BASE + jnp.minimum(sid, CH_EXTRA)
    gidx = [g0, g1, g2, g3, g4, g5, g6, g7]
    sidx = [s0, s1, s2, s3, s4, s5, s6, s7]
    rows = [r0, r1, r2, r3]
    isem = [i0, i1, i2, i3, i4, i5, i6, i7]
    gsem = [q0, q1, q2, q3]
    ssem = [w0, w1, w2, w3]

    def zrow(i, c):
        for j in range(8):
            r0[i, pl.ds(j * 16, 16)] = jnp.zeros((16,), jnp.float32)
        return c
    lax.fori_loop(0, ROW_GRAN, zrow, 0)

    def zero_chunk(i, c):
        off = pl.multiple_of((ch0 + i) * ROW_GRAN, 8)
        pltpu.sync_copy(r0, acc.at[pl.ds(off, ROW_GRAN)])
        return c
    lax.fori_loop(0, n_ch, zero_chunk, 0)
    plsc.subcore_barrier()

    def do_dir(g_hbm, s_hbm, out_off):
        ebase = sid * EPT

        def IS(e, i):
            st = ebase + i * K
            pltpu.async_copy(g_hbm.at[pl.ds(st, K)], gidx[e], isem[e])
            pltpu.async_copy(s_hbm.at[pl.ds(st, K)], sidx[e], isem[e])

        def IW(e):
            pltpu.make_async_copy(g_hbm.at[pl.ds(0, K)], gidx[e], isem[e]).wait()
            pltpu.make_async_copy(s_hbm.at[pl.ds(0, K)], sidx[e], isem[e]).wait()

        def GS(b, e):
            pltpu.async_copy(msg_hbm.at[gidx[e]], rows[b], gsem[b])

        def GW(b, e):
            pltpu.make_async_copy(msg_hbm.at[gidx[e]], rows[b], gsem[b]).wait()

        def SS(b, e):
            pltpu.async_copy(rows[b], acc.at[sidx[e]], ssem[b], add=True)

        def SW(b, e):
            pltpu.make_async_copy(rows[b], acc.at[sidx[e]], ssem[b]).wait()

        def step(t, p4, p8, clamp=False):
            SW((p4 + 2) % 4, (p8 + 6) % 8)
            c_pre = jnp.minimum(t + 4, CPT - 1) if clamp else t + 4
            IS((p8 + 4) % 8, c_pre)
            IW((p8 + 2) % 8)
            GS((p4 + 2) % 4, (p8 + 2) % 8)
            GW(p4, p8)
            SS(p4, p8)

        IS(0, 0); IS(1, 1); IS(2, 2); IS(3, 3)
        IW(0); GS(0, 0)
        IW(1); GS(1, 1)
        IS(4, 4); IW(2); GS(2, 2); GW(0, 0); SS(0, 0)
        IS(5, 5); IW(3); GS(3, 3); GW(1, 1); SS(1, 1)
        SW(0, 0); IS(6, 6); IW(4); GS(0, 4); GW(2, 2); SS(2, 2)
        SW(1, 1); IS(7, 7); IW(5); GS(1, 5); GW(3, 3); SS(3, 3)

        def octet(j, c):
            t = 8 * j + 4
            for p in range(8):
                step(t + p, p % 4, (4 + p) % 8)
            return c
        lax.fori_loop(0, 30, octet, 0)
        for u in range(244, 250):
            step(u, u % 4, u % 8, clamp=True)
        SW(0, 0); SW(1, 1)
        GW(2, 2); GW(3, 3)
        IW(4); IW(5)
        plsc.subcore_barrier()

        def out_chunk(i, c):
            off = pl.multiple_of((ch0 + i) * ROW_GRAN, 8)
            pltpu.sync_copy(acc.at[pl.ds(off, ROW_GRAN)],
                            agg_hbm.at[pl.ds(out_off + off, ROW_GRAN)])
            return c
        lax.fori_loop(0, n_ch, out_chunk, 0)

    @pl.when(cid == 0)
    def _():
        do_dir(src_hbm, dst_hbm, N)
    @pl.when(cid == 1)
    def _():
        do_dir(dstp_hbm, src_hbm, 0)


@functools.cache
def _sc_segsum():
    return pl.kernel(
        _sc_body,
        out_type=jax.ShapeDtypeStruct((2 * N, D), jnp.float32),
        mesh=plsc.VectorSubcoreMesh(core_axis_name="c", subcore_axis_name="s"),
        scratch_types=(
            [pltpu.VMEM((K,), jnp.int32) for _ in range(16)]
            + [pltpu.VMEM((K, D), jnp.float32) for _ in range(4)]
            + [pltpu.VMEM_SHARED((N, D), jnp.float32)]
            + [pltpu.SemaphoreType.DMA for _ in range(16)]
        ),
    )


def _msg_body(x_ref, w1_ref, b1_ref, w2_ref, b2_ref, o_ref):
    h = jnp.maximum(
        jnp.dot(x_ref[...], w1_ref[0], preferred_element_type=jnp.float32)
        + b1_ref[0], 0.0)
    o_ref[...] = (jnp.dot(h, w2_ref[0], preferred_element_type=jnp.float32)
                  + b2_ref[0])


_W_SPEC = pl.BlockSpec((1, D, D), lambda i: (i * 2 // G, 0, 0))
_B_SPEC = pl.BlockSpec((1, 1, D), lambda i: (i * 2 // G, 0, 0))
_X_SPEC = pl.BlockSpec((BR, D), lambda i: (i, 0))

_tc_msg = pl.pallas_call(
    _msg_body,
    grid=(G,),
    in_specs=[_X_SPEC, _W_SPEC, _B_SPEC, _W_SPEC, _B_SPEC],
    out_specs=_X_SPEC,
    out_shape=jax.ShapeDtypeStruct((2 * N, D), jnp.float32),
)


def _upd_body(a_ref, x_ref, w1a_ref, w1b_ref, b1_ref, w2_ref, b2_ref, o_ref):
    h = jnp.maximum(
        jnp.dot(a_ref[...], w1a_ref[0], preferred_element_type=jnp.float32)
        + jnp.dot(x_ref[...], w1b_ref[0], preferred_element_type=jnp.float32)
        + b1_ref[0], 0.0)
    o_ref[...] = (jnp.dot(h, w2_ref[0], preferred_element_type=jnp.float32)
                  + b2_ref[0])


def _updmsg_body(a_ref, x_ref, w1a_ref, w1b_ref, b1_ref, w2_ref, b2_ref,
                 m1_ref, c1_ref, m2_ref, c2_ref, e_ref, o_ref):
    h = jnp.maximum(
        jnp.dot(a_ref[...], w1a_ref[0], preferred_element_type=jnp.float32)
        + jnp.dot(x_ref[...], w1b_ref[0], preferred_element_type=jnp.float32)
        + b1_ref[0], 0.0)
    e = (jnp.dot(h, w2_ref[0], preferred_element_type=jnp.float32)
         + b2_ref[0])
    e_ref[...] = e
    h2 = jnp.maximum(
        jnp.dot(e, m1_ref[0], preferred_element_type=jnp.float32)
        + c1_ref[0], 0.0)
    o_ref[...] = (jnp.dot(h2, m2_ref[0], preferred_element_type=jnp.float32)
                  + c2_ref[0])


_tc_upd = pl.pallas_call(
    _upd_body,
    grid=(G,),
    in_specs=[_X_SPEC, _X_SPEC, _W_SPEC, _W_SPEC, _B_SPEC, _W_SPEC, _B_SPEC],
    out_specs=_X_SPEC,
    out_shape=jax.ShapeDtypeStruct((2 * N, D), jnp.float32),
)

_tc_updmsg = pl.pallas_call(
    _updmsg_body,
    grid=(G,),
    in_specs=[_X_SPEC, _X_SPEC, _W_SPEC, _W_SPEC, _B_SPEC, _W_SPEC, _B_SPEC,
              _W_SPEC, _B_SPEC, _W_SPEC, _B_SPEC],
    out_specs=[_X_SPEC, _X_SPEC],
    out_shape=[jax.ShapeDtypeStruct((2 * N, D), jnp.float32),
               jax.ShapeDtypeStruct((2 * N, D), jnp.float32)],
)


def kernel(v_emb, c_emb, edge_index,
           vmsg_W1, vmsg_b1, vmsg_W2, vmsg_b2,
           cmsg_W1, cmsg_b1, cmsg_W2, cmsg_b2,
           vupd_W1, vupd_b1, vupd_W2, vupd_b2,
           cupd_W1, cupd_b1, cupd_W2, cupd_b2):
    src = edge_index[0].astype(jnp.int32)
    dst = edge_index[1].astype(jnp.int32)
    dstp = dst + N

    msg_W1 = jnp.stack([vmsg_W1, cmsg_W1])
    msg_b1 = jnp.stack([vmsg_b1, cmsg_b1])[:, None, :]
    msg_W2 = jnp.stack([vmsg_W2, cmsg_W2])
    msg_b2 = jnp.stack([vmsg_b2, cmsg_b2])[:, None, :]
    upd_W1a = jnp.stack([vupd_W1[:D], cupd_W1[:D]])
    upd_W1b = jnp.stack([vupd_W1[D:], cupd_W1[D:]])
    upd_b1 = jnp.stack([vupd_b1, cupd_b1])[:, None, :]
    upd_W2 = jnp.stack([vupd_W2, cupd_W2])
    upd_b2 = jnp.stack([vupd_b2, cupd_b2])[:, None, :]

    emb = jnp.concatenate([v_emb, c_emb], axis=0)
    msg = _tc_msg(emb, msg_W1, msg_b1, msg_W2, msg_b2)
    for _ in range(NUM_ROUND - 1):
        agg = _sc_segsum()(msg, src, dst, dstp)
        emb, msg = _tc_updmsg(agg, emb, upd_W1a, upd_W1b, upd_b1, upd_W2,
                              upd_b2, msg_W1, msg_b1, msg_W2, msg_b2)
    agg = _sc_segsum()(msg, src, dst, dstp)
    emb = _tc_upd(agg, emb, upd_W1a, upd_W1b, upd_b1, upd_W2, upd_b2)
    return (emb[:N], emb[N:])
